# Initial kernel scaffold; baseline (speedup 1.0000x reference)
import jax
import jax.numpy as jnp
from jax.experimental import pallas as pl


def kernel(x, edge_index, edge_attr, pos, batch, W1a, b1a, W1b, b1b, root1, bias1, W2a, b2a, W2b, b2b, root2, bias2, fW1, fb1, fW2, fb2):
    raise NotImplementedError("write your pallas kernel here")


# speedup vs baseline: 18.1055x; 18.1055x over previous
import functools

import jax
import jax.numpy as jnp
from jax import lax
from jax.experimental import pallas as pl
from jax.experimental.pallas import tpu as pltpu
from jax.experimental.pallas import tpu_sc as plsc

_NEG_INF = float("-inf")


def _graclus_sc_body(nbr_hbm, wts_hbm, starts_hbm, out_hbm,
                     nbr_s, wts_s, starts_v, cluster_v, cbuf, wbuf):
    n = out_hbm.shape[0]
    cid = lax.axis_index("c")
    sid = lax.axis_index("s")

    @pl.when(jnp.logical_and(cid == 0, sid == 0))
    def _():
        pltpu.sync_copy(nbr_hbm, nbr_s)
        pltpu.sync_copy(wts_hbm, wts_s)
        pltpu.sync_copy(starts_hbm, starts_v.at[pl.ds(0, n + 1)])

        minus1 = jnp.full((16,), -1, jnp.int32)

        def init(i, _):
            cluster_v[pl.ds(i * 16, 16)] = minus1
            return 0
        lax.fori_loop(0, (n + 16) // 16, init, 0)

        lanes = lax.iota(jnp.int32, 16)

        int_min = jnp.int32(-2147483648)

        def node(u, _):
            ua = pl.multiple_of(u - lax.rem(u, 8), 8)
            off = u - ua
            cuv = cluster_v[pl.ds(ua, 16)]
            cu = jnp.max(jnp.where(lanes == off, cuv, int_min))

            @pl.when(cu == -1)
            def _():
                sv = starts_v[pl.ds(ua, 16)]
                s0 = jnp.max(jnp.where(lanes == off, sv, int_min))
                s1 = jnp.max(jnp.where(lanes == off + 1, sv, int_min))
                jb0 = pl.multiple_of(s0 - lax.rem(s0, 16), 16)
                nch = lax.div(s1 - jb0 + 15, 16)

                def chunk(jc, carry):
                    best, bw = carry
                    jb = pl.multiple_of(jb0 + jc * 16, 16)
                    pltpu.sync_copy(nbr_s.at[pl.ds(jb, 16)], cbuf)
                    pltpu.sync_copy(wts_s.at[pl.ds(jb, 16)], wbuf)
                    cvec = cbuf[...]
                    wvec = wbuf[...]
                    pos = lanes + jb
                    inb = jnp.logical_and(pos >= s0, pos < s1)
                    cidx = jnp.clip(cvec, 0, n - 1)
                    cl = plsc.load_gather(cluster_v, [cidx])
                    ok = jnp.logical_and(inb, cl == -1)
                    wm = jnp.where(ok, wvec, _NEG_INF)
                    m = jnp.max(wm)
                    ffs = plsc.all_reduce_ffs(wm == m)
                    cand = jnp.max(jnp.where(lanes == ffs, cvec, -1))
                    better = m > bw
                    return (jnp.where(better, cand, best),
                            jnp.where(better, m, bw))

                best, _ = lax.fori_loop(
                    0, nch, chunk, (jnp.int32(-1), jnp.float32(_NEG_INF)))
                tgt = jnp.where(best >= 0, best, u)
                widx = jnp.where(lanes == 0, u, tgt)
                plsc.store_scatter(cluster_v, [widx],
                                   jnp.full((16,), 0, jnp.int32) + u,
                                   mask=lanes < 2)
            return 0

        lax.fori_loop(0, n, node, 0)
        pltpu.sync_copy(cluster_v.at[pl.ds(0, n)], out_hbm)


def _graclus_sc(nbr, wts, starts, n):
    e2 = nbr.shape[0]
    f = pl.kernel(
        _graclus_sc_body,
        out_type=jax.ShapeDtypeStruct((n,), jnp.int32),
        mesh=plsc.VectorSubcoreMesh(core_axis_name="c", subcore_axis_name="s"),
        compiler_params=pltpu.CompilerParams(needs_layout_passes=False),
        scratch_types=[
            pltpu.VMEM_SHARED((e2,), jnp.int32),
            pltpu.VMEM_SHARED((e2,), jnp.float32),
            pltpu.VMEM((n + 16,), jnp.int32),
            pltpu.VMEM((n + 16,), jnp.int32),
            pltpu.VMEM((16,), jnp.int32),
            pltpu.VMEM((16,), jnp.float32),
        ],
    )
    return f(nbr, wts, starts)


def _graclus(r_sorted, c_sorted, w_sorted, starts, valid, n):
    wts = jnp.where(jnp.logical_and(valid, c_sorted != r_sorted),
                    w_sorted, _NEG_INF)
    pad_i = jnp.zeros((16,), jnp.int32)
    pad_f = jnp.full((16,), _NEG_INF)
    nbr = jnp.concatenate([c_sorted.astype(jnp.int32), pad_i])
    wts = jnp.concatenate([wts.astype(jnp.float32), pad_f])
    return _graclus_sc(nbr, wts, starts.astype(jnp.int32), n)


def _conv1_body(ea_ref, xg_ref, W1a_ref, b1a_ref, W1b_ref, b1b_ref, out_ref):
    z = jnp.dot(ea_ref[...], W1a_ref[...],
                preferred_element_type=jnp.float32) + b1a_ref[...]
    h = jnp.maximum(z, 0.0)
    h = jnp.dot(h, W1b_ref[...],
                preferred_element_type=jnp.float32) + b1b_ref[...]
    out_ref[...] = xg_ref[...] * h


def _conv1_msgs(ea, xg, W1a, b1a, W1b, b1b):
    E = ea.shape[0]
    B = 1280
    grid = E // B
    return pl.pallas_call(
        _conv1_body,
        grid=(grid,),
        in_specs=[
            pl.BlockSpec((B, 2), lambda i: (i, 0)),
            pl.BlockSpec((B, 1), lambda i: (i, 0)),
            pl.BlockSpec((2, 25), lambda i: (0, 0)),
            pl.BlockSpec((1, 25), lambda i: (0, 0)),
            pl.BlockSpec((25, 32), lambda i: (0, 0)),
            pl.BlockSpec((1, 32), lambda i: (0, 0)),
        ],
        out_specs=pl.BlockSpec((B, 32), lambda i: (i, 0)),
        out_shape=jax.ShapeDtypeStruct((E, 32), jnp.float32),
    )(ea, xg, W1a, b1a[None, :], W1b, b1b[None, :])


def _conv2_body(ea_ref, xs_ref, mk_ref, W2a_ref, b2a_ref, W2b_ref, Bm_ref,
                out_ref):
    z = jnp.dot(ea_ref[...], W2a_ref[...],
                preferred_element_type=jnp.float32) + b2a_ref[...]
    h = jnp.maximum(z, 0.0)
    T = jnp.dot(h, W2b_ref[...], preferred_element_type=jnp.float32)
    xs = xs_ref[...]
    msg = jnp.dot(xs, Bm_ref[...], preferred_element_type=jnp.float32)
    for i in range(32):
        msg = msg + xs[:, i:i + 1] * T[:, 64 * i:64 * (i + 1)]
    out_ref[...] = msg * mk_ref[...]


def _conv2_msgs(ea2, xs, maskf, W2a, b2a, W2b, Bm):
    E = ea2.shape[0]
    B = 640
    grid = E // B
    return pl.pallas_call(
        _conv2_body,
        grid=(grid,),
        in_specs=[
            pl.BlockSpec((B, 2), lambda i: (i, 0)),
            pl.BlockSpec((B, 32), lambda i: (i, 0)),
            pl.BlockSpec((B, 1), lambda i: (i, 0)),
            pl.BlockSpec((2, 25), lambda i: (0, 0)),
            pl.BlockSpec((1, 25), lambda i: (0, 0)),
            pl.BlockSpec((25, 2048), lambda i: (0, 0)),
            pl.BlockSpec((32, 64), lambda i: (0, 0)),
        ],
        out_specs=pl.BlockSpec((B, 64), lambda i: (i, 0)),
        out_shape=jax.ShapeDtypeStruct((E, 64), jnp.float32),
    )(ea2, xs, maskf, W2a, b2a[None, :], W2b, Bm)


def _head_body(g_ref, fW1_ref, fb1_ref, fW2_ref, fb2_ref, out_ref):
    g = g_ref[...]
    h = jnp.dot(g, fW1_ref[...],
                preferred_element_type=jnp.float32) + fb1_ref[...]
    h = jnp.where(h > 0, h, jnp.exp(jnp.minimum(h, 0.0)) - 1.0)
    o = jnp.dot(h, fW2_ref[...],
                preferred_element_type=jnp.float32) + fb2_ref[...]
    m = jnp.max(o, axis=1, keepdims=True)
    lse = jnp.log(jnp.sum(jnp.exp(o - m), axis=1, keepdims=True)) + m
    out_ref[...] = o - lse


def _head(g, fW1, fb1, fW2, fb2):
    return pl.pallas_call(
        _head_body,
        out_shape=jax.ShapeDtypeStruct((1, 10), jnp.float32),
    )(g, fW1, fb1[None, :], fW2, fb2[None, :])


def _ncut(row, col, pos, n, mask=None):
    ones = jnp.ones_like(row) if mask is None else mask.astype(row.dtype)
    d = pos[row] - pos[col]
    w = jnp.sqrt((d * d).sum(axis=1))
    deg = jnp.clip(jax.ops.segment_sum(ones, row, num_segments=n), 1).astype(pos.dtype)
    return w * (1.0 / deg[row] + 1.0 / deg[col])


def _uniq_inv(cl, n):
    present = jnp.zeros((n,), dtype=cl.dtype).at[cl].set(1)
    return jnp.cumsum(present)[cl] - 1


def _pool_struct(edge_index, pos, batch):
    n = pos.shape[0]
    row, col = edge_index[0], edge_index[1]
    pos64 = pos.astype(jnp.float64)
    w1 = _ncut(row, col, pos64, n)
    perm = jnp.argsort(row, stable=True)
    starts1 = jnp.searchsorted(row[perm], jnp.arange(n + 1))
    cl1 = _graclus(row[perm], col[perm], w1[perm], starts1,
                   jnp.ones((row.shape[0],), dtype=bool), n)
    inv1 = _uniq_inv(cl1, n)
    n2 = inv1.max() + 1
    e0 = inv1[row]; e1 = inv1[col]
    order = jnp.argsort(e0 * n + e1, stable=True)
    e0 = e0[order]; e1 = e1[order]
    keys = e0 * n + e1
    dup = jnp.concatenate([jnp.zeros((1,), dtype=bool), keys[1:] == keys[:-1]])
    emask = (e0 != e1) & ~dup
    cnt1 = jnp.clip(jax.ops.segment_sum(jnp.ones_like(inv1), inv1, num_segments=n), 1)
    pos2 = jax.ops.segment_sum(pos64, inv1, num_segments=n) / cnt1[:, None].astype(jnp.float64)
    w2 = _ncut(e0, e1, pos2, n, mask=emask)
    starts2 = jnp.searchsorted(e0, jnp.arange(n + 1))
    cl2 = _graclus(e0, e1, w2, starts2, emask, n)
    inv2 = _uniq_inv(cl2, n)
    node2 = jnp.arange(n) < n2
    n3 = jnp.where(node2, inv2, -1).max() + 1
    b1 = jnp.zeros((n,), dtype=batch.dtype).at[inv1].set(batch)
    b2 = jnp.zeros((n,), dtype=batch.dtype).at[inv2].set(b1)
    return inv1, e0, e1, emask, inv2, b2, n3


def kernel(x, edge_index, edge_attr, pos, batch, W1a, b1a, W1b, b1b, root1, bias1, W2a, b2a, W2b, b2b, root2, bias2, fW1, fb1, fW2, fb2):
    n = x.shape[0]
    E = edge_index.shape[1]
    src, dst = edge_index[0], edge_index[1]
    inv1, e0, e1, emask, inv2, b2, n3 = _pool_struct(edge_index, pos, batch)

    msg1 = _conv1_msgs(edge_attr, x[src], W1a, b1a, W1b, b1b)
    s1 = jax.ops.segment_sum(msg1, dst, num_segments=n)
    cnt_d = jnp.clip(jax.ops.segment_sum(jnp.ones(E, x.dtype), dst, num_segments=n), 1.0)
    x1 = jax.nn.elu(s1 / cnt_d[:, None] + x @ root1 + bias1)

    x1p = jax.ops.segment_max(x1, inv1, num_segments=n)
    cnt1 = jnp.clip(jax.ops.segment_sum(jnp.ones(n, dtype=x.dtype), inv1, num_segments=n), 1.0)
    pos2 = jax.ops.segment_sum(pos, inv1, num_segments=n) / cnt1[:, None]
    cart = pos2[e1] - pos2[e0]
    ea2 = cart / (2.0 * jnp.abs(cart).max()) + 0.5

    maskf = emask.astype(jnp.float32)
    msg2 = _conv2_msgs(ea2, x1p[e0], maskf[:, None], W2a, b2a, W2b,
                       b2b.reshape(32, 64))
    s2 = jax.ops.segment_sum(msg2, e1, num_segments=n)
    cnt2 = jnp.clip(jax.ops.segment_sum(maskf, e1, num_segments=n), 1.0)
    x2 = jax.nn.elu(s2 / cnt2[:, None] + x1p @ root2 + bias2)

    x2p = jax.ops.segment_max(x2, inv2, num_segments=n)
    cval = jnp.arange(n) < n3
    gcnt = jnp.clip(jax.ops.segment_sum(cval.astype(x.dtype), b2, num_segments=1), 1.0)
    g = jax.ops.segment_sum(jnp.where(cval[:, None], x2p, 0.0), b2, num_segments=1) / gcnt[:, None]
    return _head(g, fW1, fb1, fW2, fb2)


# speedup vs baseline: 34.4664x; 1.9036x over previous
import functools

import jax
import jax.numpy as jnp
from jax import lax
from jax.experimental import pallas as pl
from jax.experimental.pallas import tpu as pltpu
from jax.experimental.pallas import tpu_sc as plsc

_NEG_INF = float("-inf")


def _graclus_sc_body(nbr_hbm, wts_hbm, starts_hbm, out_hbm,
                     nbr_s, wts_s, starts_v, cluster_v, cbuf, wbuf):
    n = out_hbm.shape[0]
    cid = lax.axis_index("c")
    sid = lax.axis_index("s")

    @pl.when(jnp.logical_and(cid == 0, sid == 0))
    def _():
        pltpu.sync_copy(nbr_hbm, nbr_s)
        pltpu.sync_copy(wts_hbm, wts_s)
        pltpu.sync_copy(starts_hbm, starts_v.at[pl.ds(0, n + 1)])

        minus1 = jnp.full((16,), -1, jnp.int32)

        def init(i, _):
            cluster_v[pl.ds(i * 16, 16)] = minus1
            return 0
        lax.fori_loop(0, (n + 16) // 16, init, 0)

        lanes = lax.iota(jnp.int32, 16)

        int_min = jnp.int32(-2147483648)

        def node(u, _):
            ua = pl.multiple_of(u - lax.rem(u, 8), 8)
            off = u - ua
            cuv = cluster_v[pl.ds(ua, 16)]
            cu = jnp.max(jnp.where(lanes == off, cuv, int_min))

            @pl.when(cu == -1)
            def _():
                sv = starts_v[pl.ds(ua, 16)]
                s0 = jnp.max(jnp.where(lanes == off, sv, int_min))
                s1 = jnp.max(jnp.where(lanes == off + 1, sv, int_min))
                jb0 = pl.multiple_of(s0 - lax.rem(s0, 16), 16)
                nch = lax.div(s1 - jb0 + 15, 16)

                def chunk(jc, carry):
                    best, bw = carry
                    jb = pl.multiple_of(jb0 + jc * 16, 16)
                    pltpu.sync_copy(nbr_s.at[pl.ds(jb, 16)], cbuf)
                    pltpu.sync_copy(wts_s.at[pl.ds(jb, 16)], wbuf)
                    cvec = cbuf[...]
                    wvec = wbuf[...]
                    pos = lanes + jb
                    inb = jnp.logical_and(pos >= s0, pos < s1)
                    cidx = jnp.clip(cvec, 0, n - 1)
                    cl = plsc.load_gather(cluster_v, [cidx])
                    ok = jnp.logical_and(inb, cl == -1)
                    wm = jnp.where(ok, wvec, _NEG_INF)
                    m = jnp.max(wm)
                    ffs = plsc.all_reduce_ffs(wm == m)
                    cand = jnp.max(jnp.where(lanes == ffs, cvec, -1))
                    better = m > bw
                    return (jnp.where(better, cand, best),
                            jnp.where(better, m, bw))

                best, _ = lax.fori_loop(
                    0, nch, chunk, (jnp.int32(-1), jnp.float32(_NEG_INF)))
                tgt = jnp.where(best >= 0, best, u)
                widx = jnp.where(lanes == 0, u, tgt)
                plsc.store_scatter(cluster_v, [widx],
                                   jnp.full((16,), 0, jnp.int32) + u,
                                   mask=lanes < 2)
            return 0

        lax.fori_loop(0, n, node, 0)
        pltpu.sync_copy(cluster_v.at[pl.ds(0, n)], out_hbm)


def _graclus_sc(nbr, wts, starts, n):
    e2 = nbr.shape[0]
    f = pl.kernel(
        _graclus_sc_body,
        out_type=jax.ShapeDtypeStruct((n,), jnp.int32),
        mesh=plsc.VectorSubcoreMesh(core_axis_name="c", subcore_axis_name="s"),
        compiler_params=pltpu.CompilerParams(needs_layout_passes=False),
        scratch_types=[
            pltpu.VMEM_SHARED((e2,), jnp.int32),
            pltpu.VMEM_SHARED((e2,), jnp.float32),
            pltpu.VMEM((n + 16,), jnp.int32),
            pltpu.VMEM((n + 16,), jnp.int32),
            pltpu.VMEM((16,), jnp.int32),
            pltpu.VMEM((16,), jnp.float32),
        ],
    )
    return f(nbr, wts, starts)


def _conv1_body(ea_ref, xg_ref, W1a_ref, b1a_ref, W1b_ref, b1b_ref, out_ref):
    z = jnp.dot(ea_ref[...], W1a_ref[...],
                preferred_element_type=jnp.float32) + b1a_ref[...]
    h = jnp.maximum(z, 0.0)
    h = jnp.dot(h, W1b_ref[...],
                preferred_element_type=jnp.float32) + b1b_ref[...]
    out_ref[...] = xg_ref[...] * h


def _conv1_msgs(ea, xg, W1a, b1a, W1b, b1b):
    E = ea.shape[0]
    B = 1280
    grid = E // B
    return pl.pallas_call(
        _conv1_body,
        grid=(grid,),
        in_specs=[
            pl.BlockSpec((B, 2), lambda i: (i, 0)),
            pl.BlockSpec((B, 1), lambda i: (i, 0)),
            pl.BlockSpec((2, 25), lambda i: (0, 0)),
            pl.BlockSpec((1, 25), lambda i: (0, 0)),
            pl.BlockSpec((25, 32), lambda i: (0, 0)),
            pl.BlockSpec((1, 32), lambda i: (0, 0)),
        ],
        out_specs=pl.BlockSpec((B, 32), lambda i: (i, 0)),
        out_shape=jax.ShapeDtypeStruct((E, 32), jnp.float32),
    )(ea, xg, W1a, b1a[None, :], W1b, b1b[None, :])


def _conv2_body(ea_ref, xs_ref, mk_ref, W2a_ref, b2a_ref, W2b_ref, Bm_ref,
                out_ref):
    z = jnp.dot(ea_ref[...], W2a_ref[...],
                preferred_element_type=jnp.float32) + b2a_ref[...]
    h = jnp.maximum(z, 0.0)
    T = jnp.dot(h, W2b_ref[...], preferred_element_type=jnp.float32)
    xs = xs_ref[...]
    msg = jnp.dot(xs, Bm_ref[...], preferred_element_type=jnp.float32)
    for i in range(32):
        msg = msg + xs[:, i:i + 1] * T[:, 64 * i:64 * (i + 1)]
    out_ref[...] = msg * mk_ref[...]


def _conv2_msgs(ea2, xs, maskf, W2a, b2a, W2b, Bm):
    E = ea2.shape[0]
    B = 640
    grid = E // B
    return pl.pallas_call(
        _conv2_body,
        grid=(grid,),
        in_specs=[
            pl.BlockSpec((B, 2), lambda i: (i, 0)),
            pl.BlockSpec((B, 32), lambda i: (i, 0)),
            pl.BlockSpec((B, 1), lambda i: (i, 0)),
            pl.BlockSpec((2, 25), lambda i: (0, 0)),
            pl.BlockSpec((1, 25), lambda i: (0, 0)),
            pl.BlockSpec((25, 2048), lambda i: (0, 0)),
            pl.BlockSpec((32, 64), lambda i: (0, 0)),
        ],
        out_specs=pl.BlockSpec((B, 64), lambda i: (i, 0)),
        out_shape=jax.ShapeDtypeStruct((E, 64), jnp.float32),
    )(ea2, xs, maskf, W2a, b2a[None, :], W2b, Bm)


def _head_body(g_ref, fW1_ref, fb1_ref, fW2_ref, fb2_ref, out_ref):
    g = g_ref[...]
    h = jnp.dot(g, fW1_ref[...],
                preferred_element_type=jnp.float32) + fb1_ref[...]
    h = jnp.where(h > 0, h, jnp.exp(jnp.minimum(h, 0.0)) - 1.0)
    o = jnp.dot(h, fW2_ref[...],
                preferred_element_type=jnp.float32) + fb2_ref[...]
    m = jnp.max(o, axis=1, keepdims=True)
    lse = jnp.log(jnp.sum(jnp.exp(o - m), axis=1, keepdims=True)) + m
    out_ref[...] = o - lse


def _head(g, fW1, fb1, fW2, fb2):
    return pl.pallas_call(
        _head_body,
        out_shape=jax.ShapeDtypeStruct((1, 10), jnp.float32),
    )(g, fW1, fb1[None, :], fW2, fb2[None, :])


def _uniq_inv(cl, n):
    present = jnp.zeros((n,), dtype=cl.dtype).at[cl].set(1)
    return jnp.cumsum(present)[cl] - 1


def _pool_struct(edge_index, pos, batch):
    n = pos.shape[0]
    E = edge_index.shape[1]
    row, col = edge_index[0], edge_index[1]
    iota_e = jnp.arange(E, dtype=jnp.int32)

    deg1_i = jax.ops.segment_sum(jnp.ones_like(row), row, num_segments=n)
    invdeg1 = 1.0 / jnp.clip(deg1_i, 1).astype(jnp.float32)
    tab1 = jnp.concatenate([pos, invdeg1[:, None]], axis=1)
    gr = tab1[row]
    gc = tab1[col]
    d = gr[:, :2] - gc[:, :2]
    w1 = jnp.sqrt((d * d).sum(axis=1)) * (gr[:, 2] + gc[:, 2])

    _, perm = lax.sort((row.astype(jnp.int32), iota_e),
                       is_stable=True, num_keys=1)
    starts1 = jnp.concatenate(
        [jnp.zeros((1,), jnp.int32), jnp.cumsum(deg1_i).astype(jnp.int32)])
    wts1 = jnp.where(col != row, w1, _NEG_INF)
    packed = jnp.stack(
        [col.astype(jnp.int32), _bitcast_f2i(wts1)], axis=1)[perm]
    cl1 = _graclus_sc(
        jnp.concatenate([packed[:, 0], jnp.zeros((16,), jnp.int32)]),
        jnp.concatenate([_bitcast_i2f(packed[:, 1]),
                         jnp.full((16,), _NEG_INF, jnp.float32)]),
        starts1, n)
    inv1 = _uniq_inv(cl1, n)
    n2 = inv1.max() + 1

    e0u = inv1[row]; e1u = inv1[col]
    ks, order = lax.sort(((e0u * n + e1u).astype(jnp.int32), iota_e),
                         is_stable=True, num_keys=1)
    e0 = ks // n
    e1 = ks - e0 * n
    dup = jnp.concatenate([jnp.zeros((1,), dtype=bool), ks[1:] == ks[:-1]])
    emask = (e0 != e1) & ~dup

    cnt1 = jnp.clip(jax.ops.segment_sum(jnp.ones_like(inv1), inv1, num_segments=n), 1)
    pos2 = jax.ops.segment_sum(pos, inv1, num_segments=n) / cnt1[:, None].astype(jnp.float32)

    deg2_i = jax.ops.segment_sum(emask.astype(e0.dtype), e0, num_segments=n)
    invdeg2 = 1.0 / jnp.clip(deg2_i, 1).astype(jnp.float32)
    tab2 = jnp.concatenate([pos2, invdeg2[:, None]], axis=1)
    g0 = tab2[e0]
    g1 = tab2[e1]
    d2 = g0[:, :2] - g1[:, :2]
    w2 = jnp.sqrt((d2 * d2).sum(axis=1)) * (g0[:, 2] + g1[:, 2])
    cart = g1[:, :2] - g0[:, :2]

    deg2_all = jax.ops.segment_sum(jnp.ones_like(e0), e0, num_segments=n)
    starts2 = jnp.concatenate(
        [jnp.zeros((1,), jnp.int32), jnp.cumsum(deg2_all).astype(jnp.int32)])
    wts2 = jnp.where(emask, w2, _NEG_INF)
    cl2 = _graclus_sc(
        jnp.concatenate([e1.astype(jnp.int32), jnp.zeros((16,), jnp.int32)]),
        jnp.concatenate([wts2.astype(jnp.float32),
                         jnp.full((16,), _NEG_INF, jnp.float32)]),
        starts2, n)
    inv2 = _uniq_inv(cl2, n)
    node2 = jnp.arange(n) < n2
    n3 = jnp.where(node2, inv2, -1).max() + 1
    b1 = jnp.zeros((n,), dtype=batch.dtype).at[inv1].set(batch)
    b2 = jnp.zeros((n,), dtype=batch.dtype).at[inv2].set(b1)
    return inv1, e0, e1, emask, inv2, b2, n3, cart


def _bitcast_f2i(x):
    return lax.bitcast_convert_type(x, jnp.int32)


def _bitcast_i2f(x):
    return lax.bitcast_convert_type(x, jnp.float32)


def kernel(x, edge_index, edge_attr, pos, batch, W1a, b1a, W1b, b1b, root1, bias1, W2a, b2a, W2b, b2b, root2, bias2, fW1, fb1, fW2, fb2):
    n = x.shape[0]
    E = edge_index.shape[1]
    src, dst = edge_index[0], edge_index[1]
    inv1, e0, e1, emask, inv2, b2, n3, cart = _pool_struct(edge_index, pos, batch)

    msg1 = _conv1_msgs(edge_attr, x[src], W1a, b1a, W1b, b1b)
    s1 = jax.ops.segment_sum(msg1, dst, num_segments=n)
    cnt_d = jnp.clip(jax.ops.segment_sum(jnp.ones(E, x.dtype), dst, num_segments=n), 1.0)
    x1 = jax.nn.elu(s1 / cnt_d[:, None] + x @ root1 + bias1)

    x1p = jax.ops.segment_max(x1, inv1, num_segments=n)
    ea2 = cart / (2.0 * jnp.abs(cart).max()) + 0.5

    maskf = emask.astype(jnp.float32)
    msg2 = _conv2_msgs(ea2, x1p[e0], maskf[:, None], W2a, b2a, W2b,
                       b2b.reshape(32, 64))
    s2 = jax.ops.segment_sum(msg2, e1, num_segments=n)
    cnt2 = jnp.clip(jax.ops.segment_sum(maskf, e1, num_segments=n), 1.0)
    x2 = jax.nn.elu(s2 / cnt2[:, None] + x1p @ root2 + bias2)

    x2p = jax.ops.segment_max(x2, inv2, num_segments=n)
    cval = jnp.arange(n) < n3
    gcnt = jnp.clip(jax.ops.segment_sum(cval.astype(x.dtype), b2, num_segments=1), 1.0)
    g = jax.ops.segment_sum(jnp.where(cval[:, None], x2p, 0.0), b2, num_segments=1) / gcnt[:, None]
    return _head(g, fW1, fb1, fW2, fb2)


# speedup vs baseline: 39.2069x; 1.1375x over previous
import functools

import jax
import jax.numpy as jnp
from jax import lax
from jax.experimental import pallas as pl
from jax.experimental.pallas import tpu as pltpu
from jax.experimental.pallas import tpu_sc as plsc

_NEG_INF = float("-inf")


def _graclus_sc_body(nbr_hbm, wts_hbm, starts_hbm, out_hbm,
                     nbr_s, wts_s, starts_v, cluster_v, cbuf, wbuf,
                     sem1, sem2):
    n = out_hbm.shape[0]
    cid = lax.axis_index("c")
    sid = lax.axis_index("s")

    @pl.when(jnp.logical_and(cid == 0, sid == 0))
    def _():
        pltpu.sync_copy(nbr_hbm, nbr_s)
        pltpu.sync_copy(wts_hbm, wts_s)
        pltpu.sync_copy(starts_hbm, starts_v.at[pl.ds(0, n + 1)])

        minus1 = jnp.full((16,), -1, jnp.int32)

        def init(i, _):
            cluster_v[pl.ds(i * 16, 16)] = minus1
            return 0
        lax.fori_loop(0, (n + 16) // 16, init, 0)

        lanes = lax.iota(jnp.int32, 16)

        int_min = jnp.int32(-2147483648)

        def node(u, _):
            ua = pl.multiple_of(u - lax.rem(u, 8), 8)
            off = u - ua
            cuv = cluster_v[pl.ds(ua, 16)]
            cu = jnp.max(jnp.where(lanes == off, cuv, int_min))

            @pl.when(cu == -1)
            def _():
                sv = starts_v[pl.ds(ua, 16)]
                s0 = jnp.max(jnp.where(lanes == off, sv, int_min))
                s1 = jnp.max(jnp.where(lanes == off + 1, sv, int_min))
                jb0 = pl.multiple_of(s0 - lax.rem(s0, 16), 16)
                nch = lax.div(s1 - jb0 + 15, 16)

                def chunk(jc, carry):
                    best, bw = carry
                    jb = pl.multiple_of(jb0 + jc * 16, 16)
                    c1 = pltpu.make_async_copy(nbr_s.at[pl.ds(jb, 16)], cbuf,
                                               sem1)
                    c2 = pltpu.make_async_copy(wts_s.at[pl.ds(jb, 16)], wbuf,
                                               sem2)
                    c1.start()
                    c2.start()
                    c1.wait()
                    c2.wait()
                    cvec = cbuf[...]
                    wvec = wbuf[...]
                    pos = lanes + jb
                    inb = jnp.logical_and(pos >= s0, pos < s1)
                    cidx = jnp.clip(cvec, 0, n - 1)
                    cl = plsc.load_gather(cluster_v, [cidx])
                    ok = jnp.logical_and(inb, cl == -1)
                    wm = jnp.where(ok, wvec, _NEG_INF)
                    m = jnp.max(wm)
                    ffs = plsc.all_reduce_ffs(wm == m)
                    cand = jnp.max(jnp.where(lanes == ffs, cvec, -1))
                    better = m > bw
                    return (jnp.where(better, cand, best),
                            jnp.where(better, m, bw))

                best, _ = lax.fori_loop(
                    0, nch, chunk, (jnp.int32(-1), jnp.float32(_NEG_INF)))
                tgt = jnp.where(best >= 0, best, u)
                widx = jnp.where(lanes == 0, u, tgt)
                plsc.store_scatter(cluster_v, [widx],
                                   jnp.full((16,), 0, jnp.int32) + u,
                                   mask=lanes < 2)
            return 0

        lax.fori_loop(0, n, node, 0)
        pltpu.sync_copy(cluster_v.at[pl.ds(0, n)], out_hbm)


def _graclus_sc(nbr, wts, starts, n):
    e2 = nbr.shape[0]
    f = pl.kernel(
        _graclus_sc_body,
        out_type=jax.ShapeDtypeStruct((n,), jnp.int32),
        mesh=plsc.VectorSubcoreMesh(core_axis_name="c", subcore_axis_name="s"),
        compiler_params=pltpu.CompilerParams(needs_layout_passes=False),
        scratch_types=[
            pltpu.VMEM_SHARED((e2,), jnp.int32),
            pltpu.VMEM_SHARED((e2,), jnp.float32),
            pltpu.VMEM((n + 16,), jnp.int32),
            pltpu.VMEM((n + 16,), jnp.int32),
            pltpu.VMEM((16,), jnp.int32),
            pltpu.VMEM((16,), jnp.float32),
            pltpu.SemaphoreType.DMA,
            pltpu.SemaphoreType.DMA,
        ],
    )
    return f(nbr, wts, starts)


def _conv1_body(ea_ref, xg_ref, W1a_ref, b1a_ref, W1b_ref, b1b_ref, out_ref):
    z = jnp.dot(ea_ref[...], W1a_ref[...],
                preferred_element_type=jnp.float32) + b1a_ref[...]
    h = jnp.maximum(z, 0.0)
    h = jnp.dot(h, W1b_ref[...],
                preferred_element_type=jnp.float32) + b1b_ref[...]
    out_ref[...] = xg_ref[...] * h


def _conv1_msgs(ea, xg, W1a, b1a, W1b, b1b):
    E = ea.shape[0]
    B = 1280
    grid = E // B
    return pl.pallas_call(
        _conv1_body,
        grid=(grid,),
        in_specs=[
            pl.BlockSpec((B, 2), lambda i: (i, 0)),
            pl.BlockSpec((B, 1), lambda i: (i, 0)),
            pl.BlockSpec((2, 25), lambda i: (0, 0)),
            pl.BlockSpec((1, 25), lambda i: (0, 0)),
            pl.BlockSpec((25, 32), lambda i: (0, 0)),
            pl.BlockSpec((1, 32), lambda i: (0, 0)),
        ],
        out_specs=pl.BlockSpec((B, 32), lambda i: (i, 0)),
        out_shape=jax.ShapeDtypeStruct((E, 32), jnp.float32),
    )(ea, xg, W1a, b1a[None, :], W1b, b1b[None, :])


def _conv2_body(ea_ref, xs_ref, mk_ref, W2a_ref, b2a_ref, W2b_ref, Bm_ref,
                out_ref):
    z = jnp.dot(ea_ref[...], W2a_ref[...],
                preferred_element_type=jnp.float32) + b2a_ref[...]
    h = jnp.maximum(z, 0.0)
    T = jnp.dot(h, W2b_ref[...], preferred_element_type=jnp.float32)
    xs = xs_ref[...]
    msg = jnp.dot(xs, Bm_ref[...], preferred_element_type=jnp.float32)
    for i in range(32):
        msg = msg + xs[:, i:i + 1] * T[:, 64 * i:64 * (i + 1)]
    out_ref[...] = msg * mk_ref[...]


def _conv2_msgs(ea2, xs, maskf, W2a, b2a, W2b, Bm):
    E = ea2.shape[0]
    B = 640
    grid = E // B
    return pl.pallas_call(
        _conv2_body,
        grid=(grid,),
        in_specs=[
            pl.BlockSpec((B, 2), lambda i: (i, 0)),
            pl.BlockSpec((B, 32), lambda i: (i, 0)),
            pl.BlockSpec((B, 1), lambda i: (i, 0)),
            pl.BlockSpec((2, 25), lambda i: (0, 0)),
            pl.BlockSpec((1, 25), lambda i: (0, 0)),
            pl.BlockSpec((25, 2048), lambda i: (0, 0)),
            pl.BlockSpec((32, 64), lambda i: (0, 0)),
        ],
        out_specs=pl.BlockSpec((B, 64), lambda i: (i, 0)),
        out_shape=jax.ShapeDtypeStruct((E, 64), jnp.float32),
    )(ea2, xs, maskf, W2a, b2a[None, :], W2b, Bm)


def _head_body(g_ref, fW1_ref, fb1_ref, fW2_ref, fb2_ref, out_ref):
    g = g_ref[...]
    h = jnp.dot(g, fW1_ref[...],
                preferred_element_type=jnp.float32) + fb1_ref[...]
    h = jnp.where(h > 0, h, jnp.exp(jnp.minimum(h, 0.0)) - 1.0)
    o = jnp.dot(h, fW2_ref[...],
                preferred_element_type=jnp.float32) + fb2_ref[...]
    m = jnp.max(o, axis=1, keepdims=True)
    lse = jnp.log(jnp.sum(jnp.exp(o - m), axis=1, keepdims=True)) + m
    out_ref[...] = o - lse


def _head(g, fW1, fb1, fW2, fb2):
    return pl.pallas_call(
        _head_body,
        out_shape=jax.ShapeDtypeStruct((1, 10), jnp.float32),
    )(g, fW1, fb1[None, :], fW2, fb2[None, :])


def _uniq_inv(cl, n):
    present = jnp.zeros((n,), dtype=cl.dtype).at[cl].set(1)
    return jnp.cumsum(present)[cl] - 1


def _pool_struct(edge_index, pos, batch):
    n = pos.shape[0]
    E = edge_index.shape[1]
    row, col = edge_index[0], edge_index[1]
    iota_e = jnp.arange(E, dtype=jnp.int32)

    deg1_i = jax.ops.segment_sum(jnp.ones_like(row), row, num_segments=n)
    invdeg1 = 1.0 / jnp.clip(deg1_i, 1).astype(jnp.float32)
    tab1 = jnp.concatenate([pos, invdeg1[:, None]], axis=1)
    gr = tab1[row]
    gc = tab1[col]
    d = gr[:, :2] - gc[:, :2]
    w1 = jnp.sqrt((d * d).sum(axis=1)) * (gr[:, 2] + gc[:, 2])

    _, perm = lax.sort((row.astype(jnp.int32), iota_e),
                       is_stable=True, num_keys=1)
    starts1 = jnp.concatenate(
        [jnp.zeros((1,), jnp.int32), jnp.cumsum(deg1_i).astype(jnp.int32)])
    wts1 = jnp.where(col != row, w1, _NEG_INF)
    packed = jnp.stack(
        [col.astype(jnp.int32), _bitcast_f2i(wts1)], axis=1)[perm]
    cl1 = _graclus_sc(
        jnp.concatenate([packed[:, 0], jnp.zeros((16,), jnp.int32)]),
        jnp.concatenate([_bitcast_i2f(packed[:, 1]),
                         jnp.full((16,), _NEG_INF, jnp.float32)]),
        starts1, n)
    inv1 = _uniq_inv(cl1, n)
    n2 = inv1.max() + 1

    e0u = inv1[row]; e1u = inv1[col]
    ks, order = lax.sort(((e0u * n + e1u).astype(jnp.int32), iota_e),
                         is_stable=True, num_keys=1)
    e0 = ks // n
    e1 = ks - e0 * n
    dup = jnp.concatenate([jnp.zeros((1,), dtype=bool), ks[1:] == ks[:-1]])
    emask = (e0 != e1) & ~dup

    cnt1 = jnp.clip(jax.ops.segment_sum(jnp.ones_like(inv1), inv1, num_segments=n), 1)
    pos2 = jax.ops.segment_sum(pos, inv1, num_segments=n) / cnt1[:, None].astype(jnp.float32)

    deg2_i = jax.ops.segment_sum(emask.astype(e0.dtype), e0, num_segments=n)
    invdeg2 = 1.0 / jnp.clip(deg2_i, 1).astype(jnp.float32)
    tab2 = jnp.concatenate([pos2, invdeg2[:, None]], axis=1)
    g0 = tab2[e0]
    g1 = tab2[e1]
    d2 = g0[:, :2] - g1[:, :2]
    w2 = jnp.sqrt((d2 * d2).sum(axis=1)) * (g0[:, 2] + g1[:, 2])
    cart = g1[:, :2] - g0[:, :2]

    deg2_all = jax.ops.segment_sum(jnp.ones_like(e0), e0, num_segments=n)
    starts2 = jnp.concatenate(
        [jnp.zeros((1,), jnp.int32), jnp.cumsum(deg2_all).astype(jnp.int32)])
    wts2 = jnp.where(emask, w2, _NEG_INF)
    cl2 = _graclus_sc(
        jnp.concatenate([e1.astype(jnp.int32), jnp.zeros((16,), jnp.int32)]),
        jnp.concatenate([wts2.astype(jnp.float32),
                         jnp.full((16,), _NEG_INF, jnp.float32)]),
        starts2, n)
    inv2 = _uniq_inv(cl2, n)
    node2 = jnp.arange(n) < n2
    n3 = jnp.where(node2, inv2, -1).max() + 1
    b1 = jnp.zeros((n,), dtype=batch.dtype).at[inv1].set(batch)
    b2 = jnp.zeros((n,), dtype=batch.dtype).at[inv2].set(b1)
    return inv1, e0, e1, emask, inv2, b2, n3, cart


def _bitcast_f2i(x):
    return lax.bitcast_convert_type(x, jnp.int32)


def _bitcast_i2f(x):
    return lax.bitcast_convert_type(x, jnp.float32)


def kernel(x, edge_index, edge_attr, pos, batch, W1a, b1a, W1b, b1b, root1, bias1, W2a, b2a, W2b, b2b, root2, bias2, fW1, fb1, fW2, fb2):
    n = x.shape[0]
    E = edge_index.shape[1]
    src, dst = edge_index[0], edge_index[1]
    inv1, e0, e1, emask, inv2, b2, n3, cart = _pool_struct(edge_index, pos, batch)

    msg1 = _conv1_msgs(edge_attr, x[src], W1a, b1a, W1b, b1b)
    s1 = jax.ops.segment_sum(msg1, dst, num_segments=n)
    cnt_d = jnp.clip(jax.ops.segment_sum(jnp.ones(E, x.dtype), dst, num_segments=n), 1.0)
    x1 = jax.nn.elu(s1 / cnt_d[:, None] + x @ root1 + bias1)

    x1p = jax.ops.segment_max(x1, inv1, num_segments=n)
    ea2 = cart / (2.0 * jnp.abs(cart).max()) + 0.5

    maskf = emask.astype(jnp.float32)
    msg2 = _conv2_msgs(ea2, x1p[e0], maskf[:, None], W2a, b2a, W2b,
                       b2b.reshape(32, 64))
    s2 = jax.ops.segment_sum(msg2, e1, num_segments=n)
    cnt2 = jnp.clip(jax.ops.segment_sum(maskf, e1, num_segments=n), 1.0)
    x2 = jax.nn.elu(s2 / cnt2[:, None] + x1p @ root2 + bias2)

    x2p = jax.ops.segment_max(x2, inv2, num_segments=n)
    cval = jnp.arange(n) < n3
    gcnt = jnp.clip(jax.ops.segment_sum(cval.astype(x.dtype), b2, num_segments=1), 1.0)
    g = jax.ops.segment_sum(jnp.where(cval[:, None], x2p, 0.0), b2, num_segments=1) / gcnt[:, None]
    return _head(g, fW1, fb1, fW2, fb2)
